# Initial kernel scaffold; baseline (speedup 1.0000x reference)
import jax
import jax.numpy as jnp
from jax.experimental import pallas as pl


def kernel(full_feature, corr_feature, anchor):
    raise NotImplementedError("write your pallas kernel here")


# speedup vs baseline: 3.3500x; 3.3500x over previous
import functools

import jax
import jax.numpy as jnp
from jax import lax
from jax.experimental import pallas as pl
from jax.experimental.pallas import tpu as pltpu
from jax.experimental.pallas import tpu_sc as plsc

STRIDE = 4
PATCH = 31
B = 4
A = 8
C = 256
H = 127
W = 127
HC = 25
WC = 25
L = 16
CCH = 4
NCH = C // CCH
NWID = B * A
ROWS = 40


def _tc_softmax_body(anc_s, corr_ref, w3_ref):
    ba = pl.program_id(0) * A + pl.program_id(1)
    y = anc_s[2 * ba]
    x = anc_s[2 * ba + 1]
    row = corr_ref[0, :, pl.ds(y, 1), :].reshape(C, WC)
    onehot = (lax.broadcasted_iota(jnp.int32, (C, WC), 1) == x)
    vec = jnp.sum(jnp.where(onehot, row, 0.0), axis=1).reshape(2, 128)
    m = jnp.max(vec)
    e = jnp.exp(vec - m)
    w3_ref[0, pl.ds(0, 2), :] = e / jnp.sum(e)


def _tc_softmax(corr_feature, anchor_flat):
    grid_spec = pltpu.PrefetchScalarGridSpec(
        num_scalar_prefetch=1,
        grid=(B, A),
        in_specs=[
            pl.BlockSpec((1, C, HC, WC), lambda b, a, s: (b, 0, 0, 0)),
        ],
        out_specs=pl.BlockSpec((1, 8, 128), lambda b, a, s: (b * A + a, 0, 0)),
    )
    return pl.pallas_call(
        _tc_softmax_body,
        grid_spec=grid_spec,
        out_shape=jax.ShapeDtypeStruct((NWID, 8, 128), jnp.float32),
    )(anchor_flat, corr_feature)


def _sc_body(full_ref, w3_ref, anc_ref, out_ref,
             anc_v, w_v, buf_v, obuf_v,
             in_sem0, in_sem1, out_sem0, out_sem1):
    in_sems = (in_sem0, in_sem1)
    out_sems = (out_sem0, out_sem1)

    wid = lax.axis_index("s") * 2 + lax.axis_index("c")
    b = wid // A
    a = wid % A

    pltpu.sync_copy(anc_ref, anc_v)
    basev = jnp.full((L,), 2 * wid, dtype=jnp.int32)
    y = jnp.max(plsc.load_gather(anc_v, [basev]))
    x = jnp.max(plsc.load_gather(anc_v, [basev + 1]))
    y4 = y * STRIDE
    x4 = x * STRIDE
    ybase = pl.multiple_of(jnp.minimum(y4 & ~7, 88), 8)
    d = y4 - ybase

    pltpu.sync_copy(w3_ref.at[wid], w_v)

    def start_in(g, bi):
        src = full_ref.at[b, pl.ds(g * CCH, CCH), pl.ds(ybase, ROWS), :]
        return pltpu.async_copy(src, buf_v.at[bi], in_sems[bi])

    def start_out(g, bi):
        dst = out_ref.at[b, a, pl.ds(g * CCH, CCH)]
        return pltpu.async_copy(obuf_v.at[bi], dst, out_sems[bi])

    def scale_chunk(g, bi):
        def cc_body(cc, carry):
            cglob = g * CCH + cc
            wsplat = plsc.load_gather(
                w_v, [jnp.full((L,), cglob // 128, dtype=jnp.int32),
                      jnp.full((L,), cglob % 128, dtype=jnp.int32)])

            def r_body(r, inner):
                v0 = buf_v[bi, cc, d + r, pl.ds(x4, L)]
                v1 = buf_v[bi, cc, d + r, pl.ds(x4 + PATCH - L, L)]
                obuf_v[bi, cc, r, pl.ds(0, L)] = v0 * wsplat
                obuf_v[bi, cc, r, pl.ds(PATCH - L, L)] = v1 * wsplat
                return inner

            return lax.fori_loop(0, PATCH, r_body, carry)

        lax.fori_loop(0, CCH, cc_body, 0)

    in_descs = [None] * NCH
    out_descs = [None] * NCH
    in_descs[0] = start_in(0, 0)
    for g in range(NCH):
        bi = g % 2
        if g + 1 < NCH:
            if g >= 1:
                out_descs[g - 1].wait()
            in_descs[g + 1] = start_in(g + 1, 1 - bi)
        in_descs[g].wait()
        scale_chunk(g, bi)
        out_descs[g] = start_out(g, bi)
    out_descs[NCH - 2].wait()
    out_descs[NCH - 1].wait()


@jax.jit
def _run(full_feature, corr_feature, anchor_flat):
    w3 = _tc_softmax(corr_feature, anchor_flat)
    mesh = plsc.VectorSubcoreMesh(core_axis_name="c", subcore_axis_name="s")
    fn = pl.kernel(
        _sc_body,
        out_type=jax.ShapeDtypeStruct((B, A, C, 32, 128), jnp.float32),
        mesh=mesh,
        compiler_params=pltpu.CompilerParams(
            use_tc_tiling_on_sc=True, needs_layout_passes=False),
        scratch_types=[
            pltpu.VMEM((2 * NWID,), jnp.int32),
            pltpu.VMEM((8, 128), jnp.float32),
            pltpu.VMEM((2, CCH, ROWS, W), jnp.float32),
            pltpu.VMEM((2, CCH, 32, 128), jnp.float32),
            pltpu.SemaphoreType.DMA,
            pltpu.SemaphoreType.DMA,
            pltpu.SemaphoreType.DMA,
            pltpu.SemaphoreType.DMA,
        ],
    )
    out_padded = fn(full_feature, w3, anchor_flat)
    return out_padded[:, :, :, :PATCH, :PATCH]


def kernel(full_feature, corr_feature, anchor):
    anchor_flat = anchor.reshape(-1).astype(jnp.int32)
    return _run(full_feature, corr_feature, anchor_flat)


# speedup vs baseline: 12.5367x; 3.7422x over previous
import functools

import jax
import jax.numpy as jnp
from jax import lax
from jax.experimental import pallas as pl
from jax.experimental.pallas import tpu as pltpu
from jax.experimental.pallas import tpu_sc as plsc

STRIDE = 4
PATCH = 31
B = 4
A = 8
C = 256
H = 127
W = 127
HC = 25
WC = 25
L = 16
CH = C // 128
NROW = 4
NSLOT = NROW


def _sc_body(full_rows, corr_rows, anc_ref, out_rows,
             anc_v, crow_v, wbuf_v, gbuf_v,
             csem, gsem0, gsem1, gsem2, gsem3,
             osem0, osem1, osem2, osem3):
    gsems = (gsem0, gsem1, gsem2, gsem3)
    osems = (osem0, osem1, osem2, osem3)

    wid = lax.axis_index("s") * 2 + lax.axis_index("c")
    b = wid // A
    ig = wid % A
    i0 = ig * NROW

    lanes = lax.iota(jnp.int32, L)
    av = lanes & 7
    chv = lanes >> 3

    pltpu.sync_copy(anc_ref, anc_v)
    ay = plsc.load_gather(anc_v, [(b * A + av) * 2])
    ax = plsc.load_gather(anc_v, [(b * A + av) * 2 + 1])

    cidx = ((ay * WC + ax) * CH + chv) * B + b
    pltpu.async_copy(corr_rows.at[cidx], crow_v, csem).wait()

    for a in range(A):
        m = jnp.maximum(crow_v[a, pl.ds(0, L)], crow_v[a + A, pl.ds(0, L)])
        for k in range(1, 128 // L):
            m = jnp.maximum(m, crow_v[a, pl.ds(k * L, L)])
            m = jnp.maximum(m, crow_v[a + A, pl.ds(k * L, L)])
        mm = jnp.max(m)
        s = jnp.zeros((L,), jnp.float32)
        for row in (a, a + A):
            for k in range(128 // L):
                e = jnp.exp(crow_v[row, pl.ds(k * L, L)] - mm)
                wbuf_v[row, pl.ds(k * L, L)] = e
                s = s + e
        invv = 1.0 / jnp.full((L,), jnp.sum(s), dtype=jnp.float32)
        for row in (a, a + A):
            for k in range(128 // L):
                wbuf_v[row, pl.ds(k * L, L)] = (
                    wbuf_v[row, pl.ds(k * L, L)] * invv)

    hbase = ay * STRIDE
    wbase = ax * STRIDE

    def gather_idx(i, j):
        hv = jnp.minimum(hbase + i, H - 1)
        wv = wbase + j
        return ((hv * W + wv) * CH + chv) * B + b

    def out_base(i, j):
        return ((b * PATCH + i) * PATCH + j) * L

    def scale_slot(s):
        def l_body(l, carry):
            for k in range(128 // L):
                gbuf_v[s, l, pl.ds(k * L, L)] = (
                    gbuf_v[s, l, pl.ds(k * L, L)]
                    * wbuf_v[l, pl.ds(k * L, L)])
            return carry
        lax.fori_loop(0, L, l_body, 0)

    def j_body(j, carry):
        for s in range(NSLOT):
            i = i0 + s

            @pl.when(jnp.logical_and(j > 0, i < PATCH))
            def _wait_prev_out():
                pltpu.make_async_copy(
                    gbuf_v.at[s], out_rows.at[pl.ds(0, L)],
                    osems[s]).wait()

            pltpu.async_copy(full_rows.at[gather_idx(i, j)],
                             gbuf_v.at[s], gsems[s])
        for s in range(NSLOT):
            i = i0 + s
            pltpu.make_async_copy(full_rows.at[pl.ds(0, L)],
                                  gbuf_v.at[s], gsems[s]).wait()
            scale_slot(s)

            @pl.when(i < PATCH)
            def _store_out():
                pltpu.async_copy(
                    gbuf_v.at[s],
                    out_rows.at[pl.ds(out_base(i, j), L)], osems[s])
        return carry

    lax.fori_loop(0, PATCH, j_body, 0)

    for s in range(NSLOT):
        @pl.when(i0 + s < PATCH)
        def _drain():
            pltpu.make_async_copy(
                gbuf_v.at[s], out_rows.at[pl.ds(0, L)], osems[s]).wait()


@jax.jit
def _run(full_feature, corr_feature, anchor_flat):
    full_rows = full_feature.reshape(B, CH, 128, H, W).transpose(
        3, 4, 1, 0, 2).reshape(H * W * CH * B, 128)
    corr_rows = corr_feature.reshape(B, CH, 128, HC, WC).transpose(
        3, 4, 1, 0, 2).reshape(HC * WC * CH * B, 128)

    mesh = plsc.VectorSubcoreMesh(core_axis_name="c", subcore_axis_name="s")
    fn = pl.kernel(
        _sc_body,
        out_type=jax.ShapeDtypeStruct((B * PATCH * PATCH * L, 128),
                                      jnp.float32),
        mesh=mesh,
        compiler_params=pltpu.CompilerParams(
            use_tc_tiling_on_sc=False, needs_layout_passes=False),
        scratch_types=[
            pltpu.VMEM((2 * B * A,), jnp.int32),
            pltpu.VMEM((L, 128), jnp.float32),
            pltpu.VMEM((L, 128), jnp.float32),
            pltpu.VMEM((NSLOT, L, 128), jnp.float32),
        ] + [pltpu.SemaphoreType.DMA] * 9,
    )
    out_rows = fn(full_rows, corr_rows, anchor_flat)
    return out_rows.reshape(B, PATCH, PATCH, CH, A, 128).transpose(
        0, 4, 3, 5, 1, 2).reshape(B, A, C, PATCH, PATCH)


def kernel(full_feature, corr_feature, anchor):
    anchor_flat = anchor.reshape(-1).astype(jnp.int32)
    return _run(full_feature, corr_feature, anchor_flat)


# speedup vs baseline: 13.6758x; 1.0909x over previous
import functools

import jax
import jax.numpy as jnp
from jax import lax
from jax.experimental import pallas as pl
from jax.experimental.pallas import tpu as pltpu
from jax.experimental.pallas import tpu_sc as plsc

STRIDE = 4
PATCH = 31
B = 4
A = 8
C = 256
H = 127
W = 127
HC = 25
WC = 25
L = 16
CH = C // 128
NROW = 4
NCOL = 2
NSLOT = NROW * NCOL
NJ = (PATCH + NCOL - 1) // NCOL


def _sc_body(full_rows, corr_rows, anc_ref, out_rows,
             anc_v, crow_v, wbuf_v, gbuf_v,
             csem,
             gsem0, gsem1, gsem2, gsem3, gsem4, gsem5, gsem6, gsem7,
             osem0, osem1, osem2, osem3, osem4, osem5, osem6, osem7):
    gsems = (gsem0, gsem1, gsem2, gsem3, gsem4, gsem5, gsem6, gsem7)
    osems = (osem0, osem1, osem2, osem3, osem4, osem5, osem6, osem7)

    wid = lax.axis_index("s") * 2 + lax.axis_index("c")
    b = wid // A
    ig = wid % A
    i0 = ig * NROW

    lanes = lax.iota(jnp.int32, L)
    av = lanes & 7
    chv = lanes >> 3

    pltpu.sync_copy(anc_ref, anc_v)
    ay = plsc.load_gather(anc_v, [(b * A + av) * 2])
    ax = plsc.load_gather(anc_v, [(b * A + av) * 2 + 1])

    cidx = ((ay * WC + ax) * CH + chv) * B + b
    pltpu.async_copy(corr_rows.at[cidx], crow_v, csem).wait()

    for a in range(A):
        m = jnp.maximum(crow_v[a, pl.ds(0, L)], crow_v[a + A, pl.ds(0, L)])
        for k in range(1, 128 // L):
            m = jnp.maximum(m, crow_v[a, pl.ds(k * L, L)])
            m = jnp.maximum(m, crow_v[a + A, pl.ds(k * L, L)])
        mm = jnp.max(m)
        s = jnp.zeros((L,), jnp.float32)
        for row in (a, a + A):
            for k in range(128 // L):
                e = jnp.exp(crow_v[row, pl.ds(k * L, L)] - mm)
                wbuf_v[row, pl.ds(k * L, L)] = e
                s = s + e
        invv = 1.0 / jnp.full((L,), jnp.sum(s), dtype=jnp.float32)
        for row in (a, a + A):
            for k in range(128 // L):
                wbuf_v[row, pl.ds(k * L, L)] = (
                    wbuf_v[row, pl.ds(k * L, L)] * invv)

    hbase = ay * STRIDE
    wbase = ax * STRIDE

    def gather_idx(i, j):
        hv = jnp.minimum(hbase + i, H - 1)
        wv = jnp.minimum(wbase + j, W - 1)
        return ((hv * W + wv) * CH + chv) * B + b

    def out_base(i, j):
        return ((b * PATCH + i) * PATCH + j) * L

    def scale_all():
        def l_body(l, carry):
            wrow = [wbuf_v[l, pl.ds(k * L, L)] for k in range(128 // L)]
            for s in range(NSLOT):
                for k in range(128 // L):
                    gbuf_v[s, l, pl.ds(k * L, L)] = (
                        gbuf_v[s, l, pl.ds(k * L, L)] * wrow[k])
            return carry
        lax.fori_loop(0, L, l_body, 0)

    def jj_body(jj, carry):
        for s in range(NSLOT):
            i = i0 + s % NROW
            j = jj * NCOL + s // NROW
            @pl.when(jnp.logical_and(jj > 0, i < PATCH))
            def _wait_prev_out():
                pltpu.make_async_copy(
                    gbuf_v.at[s], out_rows.at[pl.ds(0, L)],
                    osems[s]).wait()

            pltpu.async_copy(full_rows.at[gather_idx(i, j)],
                             gbuf_v.at[s], gsems[s])
        for s in range(NSLOT):
            pltpu.make_async_copy(full_rows.at[pl.ds(0, L)],
                                  gbuf_v.at[s], gsems[s]).wait()
        scale_all()
        for s in range(NSLOT):
            i = i0 + s % NROW
            j = jj * NCOL + s // NROW
            valid = jnp.logical_and(i < PATCH, j < PATCH)

            @pl.when(valid)
            def _store_out():
                pltpu.async_copy(
                    gbuf_v.at[s],
                    out_rows.at[pl.ds(out_base(i, j), L)], osems[s])
        return carry

    lax.fori_loop(0, NJ, jj_body, 0)

    for s in range(NSLOT):
        i = i0 + s % NROW
        j = (NJ - 1) * NCOL + s // NROW
        valid = jnp.logical_and(i < PATCH, j < PATCH)

        @pl.when(valid)
        def _drain():
            pltpu.make_async_copy(
                gbuf_v.at[s], out_rows.at[pl.ds(0, L)], osems[s]).wait()


@jax.jit
def _run(full_feature, corr_feature, anchor_flat):
    full_rows = full_feature.reshape(B, CH, 128, H, W).transpose(
        3, 4, 1, 0, 2).reshape(H * W * CH * B, 128)
    corr_rows = corr_feature.reshape(B, CH, 128, HC, WC).transpose(
        3, 4, 1, 0, 2).reshape(HC * WC * CH * B, 128)

    mesh = plsc.VectorSubcoreMesh(core_axis_name="c", subcore_axis_name="s")
    fn = pl.kernel(
        _sc_body,
        out_type=jax.ShapeDtypeStruct((B * PATCH * PATCH * L, 128),
                                      jnp.float32),
        mesh=mesh,
        compiler_params=pltpu.CompilerParams(
            use_tc_tiling_on_sc=False, needs_layout_passes=False),
        scratch_types=[
            pltpu.VMEM((2 * B * A,), jnp.int32),
            pltpu.VMEM((L, 128), jnp.float32),
            pltpu.VMEM((L, 128), jnp.float32),
            pltpu.VMEM((NSLOT, L, 128), jnp.float32),
        ] + [pltpu.SemaphoreType.DMA] * 17,
    )
    out_rows = fn(full_rows, corr_rows, anchor_flat)
    return out_rows.reshape(B, PATCH, PATCH, CH, A, 128).transpose(
        0, 4, 3, 5, 1, 2).reshape(B, A, C, PATCH, PATCH)


def kernel(full_feature, corr_feature, anchor):
    anchor_flat = anchor.reshape(-1).astype(jnp.int32)
    return _run(full_feature, corr_feature, anchor_flat)
